# baseline (speedup 1.0000x reference)
import functools
import math

import jax
import jax.numpy as jnp
from jax import lax
from jax.experimental import pallas as pl
from jax.experimental.pallas import tpu as pltpu
from jax.experimental.pallas import tpu_sc as plsc

D_EMB = 64
SCALE = math.sqrt(D_EMB)

_info = plsc.get_sparse_core_info()
_NC, _NS, _L = _info.num_cores, _info.num_subcores, _info.num_lanes
_NW = _NC * _NS


def _make_gather(B: int, V: int, D: int, C: int):
  assert B % (_NW * C) == 0 and C % 8 == 0 and D % _L == 0
  b_per_w = B // _NW
  n_chunks = b_per_w // C
  mesh = plsc.VectorSubcoreMesh(core_axis_name="c", subcore_axis_name="s")

  @functools.partial(
      pl.kernel,
      mesh=mesh,
      out_type=jax.ShapeDtypeStruct((B, D), jnp.float32),
      compiler_params=pltpu.CompilerParams(use_tc_tiling_on_sc=False),
      scratch_types=[
          pltpu.VMEM((C,), jnp.int32),
          pltpu.VMEM((C, D), jnp.float32),
          pltpu.SemaphoreType.DMA,
      ],
  )
  def gather_kernel(table_hbm, idx_hbm, out_hbm, idx_v, rows_v, sem):
    wid = lax.axis_index("s") * _NC + lax.axis_index("c")
    base = wid * b_per_w

    def chunk_body(g, carry):
      off = base + g * C
      pltpu.sync_copy(idx_hbm.at[pl.ds(off, C)], idx_v)
      pltpu.async_copy(table_hbm.at[idx_v], rows_v, sem).wait()

      def scale_row(j, c2):
        for k in range(D // _L):
          sl = pl.ds(k * _L, _L)
          rows_v[j, sl] = rows_v[j, sl] * SCALE
        return c2

      lax.fori_loop(0, C, scale_row, 0)
      pltpu.sync_copy(rows_v, out_hbm.at[pl.ds(off, C)])
      return carry

    lax.fori_loop(0, n_chunks, chunk_body, 0)

  return gather_kernel


def kernel(seq, table):
  bsz, hist = seq.shape
  B = bsz * hist
  V, D = table.shape
  idx = seq.reshape(B)
  out = _make_gather(B, V, D, C=512)(table, idx)
  return out.reshape(bsz, hist, D)
